# baseline (speedup 1.0000x reference)
import jax
import jax.numpy as jnp
from jax import lax
from jax.experimental import pallas as pl
from jax.experimental.pallas import tpu as pltpu

_M = 65536
_B = 1024
_D = 64
_BLK = 2048
_STEPS = _M // _BLK
_HI = lax.Precision.HIGHEST


def _loss_body(emb_ref, epad_ref, labc_ref, labr_ref, mem_ref, labm_ref,
               out_ref, sums, cnts):
    pi = pl.program_id(0)

    @pl.when(pi == 0)
    def _init():
        sums[0] = 0.0
        sums[1] = 0.0
        cnts[0] = 0
        cnts[1] = 0

    q = emb_ref[...]
    qs = jnp.sum(q * q, axis=1, keepdims=True)
    qn = q * (1.0 / jnp.maximum(jnp.sqrt(qs), 1e-12))
    qq = jnp.sum(qn * qn, axis=1, keepdims=True)

    rows = lax.broadcasted_iota(jnp.int32, (_BLK, 1), 0) + pi * _BLK
    r = jnp.where(rows < _B, epad_ref[...], mem_ref[...])
    rs = jnp.sum(r * r, axis=1, keepdims=True)
    rn = r * (1.0 / jnp.maximum(jnp.sqrt(rs), 1e-12))

    dots = lax.dot_general(qn, rn, (((1,), (1,)), ((), ())),
                           precision=_HI,
                           preferred_element_type=jnp.float32)
    rr_row = lax.dot_general(jnp.ones((1, _D), jnp.float32), rn * rn,
                             (((1,), (1,)), ((), ())),
                             precision=_HI,
                             preferred_element_type=jnp.float32)

    sq = qq + rr_row - 2.0 * dots
    dist = jnp.sqrt(jnp.maximum(sq, 1e-12))

    cols = lax.broadcasted_iota(jnp.int32, (1, _BLK), 1) + pi * _BLK
    labsrc = jnp.where(cols < _B, labr_ref[0], labm_ref[0])
    m = labc_ref[...] == labsrc
    ii = lax.broadcasted_iota(jnp.int32, (_B, 1), 0)
    posm = m & (ii != cols)

    pos_v = jnp.where(posm, dist, 0.0)
    neg_v = jnp.where(m, 0.0, jnp.maximum(1.0 - dist, 0.0))

    sums[0] += jnp.sum(pos_v)
    sums[1] += jnp.sum(neg_v)
    cnts[0] += jnp.sum(posm.astype(jnp.int32))
    cnts[1] += jnp.sum((neg_v > 0.0).astype(jnp.int32))

    @pl.when(pi == _STEPS - 1)
    def _fin():
        pc = jnp.maximum(cnts[0], 1).astype(jnp.float32)
        nc = jnp.maximum(cnts[1], 1).astype(jnp.float32)
        loss = (sums[0] / pc + sums[1] / nc) * 0.5
        out_ref[...] = jnp.full((1, 1), loss, jnp.float32)


def kernel(embeddings, labels, embedding_memory, label_memory):
    labels = labels.astype(jnp.int32)
    label_memory = label_memory.astype(jnp.int32)

    if _BLK == _B:
        epad = embeddings
        labr = labels
    else:
        epad = jnp.zeros((_BLK, _D), jnp.float32).at[:_B].set(embeddings)
        labr = jnp.zeros((_BLK,), jnp.int32).at[:_B].set(labels)
    labr3 = labr.reshape(1, 1, _BLK)
    labc = labels.reshape(_B, 1)
    labm3 = label_memory.reshape(_STEPS, 1, _BLK)

    out = pl.pallas_call(
        _loss_body,
        grid=(_STEPS,),
        in_specs=[
            pl.BlockSpec((_B, _D), lambda i: (0, 0)),
            pl.BlockSpec((_BLK, _D), lambda i: (0, 0)),
            pl.BlockSpec((_B, 1), lambda i: (0, 0)),
            pl.BlockSpec((1, 1, _BLK), lambda i: (0, 0, 0)),
            pl.BlockSpec((_BLK, _D), lambda i: (i, 0)),
            pl.BlockSpec((1, 1, _BLK), lambda i: (i, 0, 0)),
        ],
        out_specs=pl.BlockSpec((1, 1), lambda i: (0, 0)),
        out_shape=jax.ShapeDtypeStruct((1, 1), jnp.float32),
        scratch_shapes=[
            pltpu.SMEM((2,), jnp.float32),
            pltpu.SMEM((2,), jnp.int32),
        ],
        compiler_params=pltpu.CompilerParams(
            dimension_semantics=("arbitrary",)),
    )(embeddings, epad, labc, labr3, embedding_memory, labm3)
    return out[0, 0]


# speedup vs baseline: 1.4561x; 1.4561x over previous
import jax
import jax.numpy as jnp
from jax import lax
from jax.experimental import pallas as pl
from jax.experimental.pallas import tpu as pltpu

_M = 65536
_B = 1024
_D = 64
_BLK = 2048
_STEPS = _M // _BLK
_HI = lax.Precision.HIGHEST


def _normalize_rows(x):
    s = jnp.sum(x * x, axis=1, keepdims=True)
    return x * lax.rsqrt(jnp.maximum(s, 1e-24))


def _block_terms(qn, rn, labc, labsrc):
    dots = lax.dot_general(qn, rn, (((1,), (1,)), ((), ())),
                           precision=_HI,
                           preferred_element_type=jnp.float32)
    dist = jnp.sqrt(jnp.maximum(2.0 - 2.0 * dots, 1e-12))
    m = labc == labsrc
    neg_v = jnp.where(m, 0.0, jnp.maximum(1.0 - dist, 0.0))
    nn = (~m) & (dots > 0.5)
    return dist, m, neg_v, nn


def _loss_body(emb_ref, epad_ref, labc_ref, labr_ref, mem_ref, labm_ref,
               out_ref, qn_ref, sums, cnts):
    pi = pl.program_id(0)
    labc = labc_ref[...]

    @pl.when(pi == 0)
    def _first():
        sums[0] = 0.0
        sums[1] = 0.0
        cnts[0] = 0
        cnts[1] = 0
        qn_ref[...] = _normalize_rows(emb_ref[...])
        rows = lax.broadcasted_iota(jnp.int32, (_BLK, 1), 0)
        r = jnp.where(rows < _B, epad_ref[...], mem_ref[...])
        rn = _normalize_rows(r)
        cols = lax.broadcasted_iota(jnp.int32, (1, _BLK), 1)
        labsrc = jnp.where(cols < _B, labr_ref[0], labm_ref[0])
        dist, m, neg_v, nn = _block_terms(qn_ref[...], rn, labc, labsrc)
        ii = lax.broadcasted_iota(jnp.int32, (_B, 1), 0)
        posm = m & (ii != cols)
        sums[0] += jnp.sum(jnp.where(posm, dist, 0.0))
        sums[1] += jnp.sum(neg_v)
        cnts[0] += jnp.sum(posm.astype(jnp.int32))
        cnts[1] += jnp.sum(nn.astype(jnp.int32))

    @pl.when(pi != 0)
    def _rest():
        rn = _normalize_rows(mem_ref[...])
        dist, m, neg_v, nn = _block_terms(qn_ref[...], rn, labc, labm_ref[0])
        sums[0] += jnp.sum(jnp.where(m, dist, 0.0))
        sums[1] += jnp.sum(neg_v)
        cnts[0] += jnp.sum(m.astype(jnp.int32))
        cnts[1] += jnp.sum(nn.astype(jnp.int32))

    @pl.when(pi == _STEPS - 1)
    def _fin():
        pc = jnp.maximum(cnts[0], 1).astype(jnp.float32)
        nc = jnp.maximum(cnts[1], 1).astype(jnp.float32)
        loss = (sums[0] / pc + sums[1] / nc) * 0.5
        out_ref[...] = jnp.full((1, 1), loss, jnp.float32)


def kernel(embeddings, labels, embedding_memory, label_memory):
    labels = labels.astype(jnp.int32)
    label_memory = label_memory.astype(jnp.int32)

    epad = jnp.zeros((_BLK, _D), jnp.float32).at[:_B].set(embeddings)
    labr = jnp.zeros((_BLK,), jnp.int32).at[:_B].set(labels)
    labr3 = labr.reshape(1, 1, _BLK)
    labc = labels.reshape(_B, 1)
    labm3 = label_memory.reshape(_STEPS, 1, _BLK)

    out = pl.pallas_call(
        _loss_body,
        grid=(_STEPS,),
        in_specs=[
            pl.BlockSpec((_B, _D), lambda i: (0, 0)),
            pl.BlockSpec((_BLK, _D), lambda i: (0, 0)),
            pl.BlockSpec((_B, 1), lambda i: (0, 0)),
            pl.BlockSpec((1, 1, _BLK), lambda i: (0, 0, 0)),
            pl.BlockSpec((_BLK, _D), lambda i: (i, 0)),
            pl.BlockSpec((1, 1, _BLK), lambda i: (i, 0, 0)),
        ],
        out_specs=pl.BlockSpec((1, 1), lambda i: (0, 0)),
        out_shape=jax.ShapeDtypeStruct((1, 1), jnp.float32),
        scratch_shapes=[
            pltpu.VMEM((_B, _D), jnp.float32),
            pltpu.SMEM((2,), jnp.float32),
            pltpu.SMEM((2,), jnp.int32),
        ],
        compiler_params=pltpu.CompilerParams(
            dimension_semantics=("arbitrary",)),
    )(embeddings, epad, labc, labr3, embedding_memory, labm3)
    return out[0, 0]


# speedup vs baseline: 1.5065x; 1.0346x over previous
import jax
import jax.numpy as jnp
from jax import lax
from jax.experimental import pallas as pl
from jax.experimental.pallas import tpu as pltpu

_M = 65536
_B = 1024
_D = 64
_BLK = 2048
_STEPS = _M // _BLK
_HI = lax.Precision.HIGHEST


def _normalize_rows(x):
    s = jnp.sum(x * x, axis=1, keepdims=True)
    return x * lax.rsqrt(jnp.maximum(s, 1e-24))


def _block_terms(qn, rn, labc, labsrc):
    dots = lax.dot_general(qn, rn, (((1,), (1,)), ((), ())),
                           precision=_HI,
                           preferred_element_type=jnp.float32)
    sq = lax.max(2.0 - 2.0 * dots, 1e-12)
    dist = sq * lax.rsqrt(sq)
    m = labc == labsrc
    nn = (~m) & (dots > 0.5)
    nn_v = jnp.where(nn, dist, 0.0)
    return dist, m, nn_v, nn


def _loss_body(emb_ref, epad_ref, labc_ref, labr_ref, mem_ref, labm_ref,
               out_ref, qn_ref, sums, cnts):
    pi = pl.program_id(0)
    labc = labc_ref[...]

    @pl.when(pi == 0)
    def _first():
        sums[0] = 0.0
        sums[1] = 0.0
        cnts[0] = 0
        cnts[1] = 0
        qn_ref[...] = _normalize_rows(emb_ref[...])
        rows = lax.broadcasted_iota(jnp.int32, (_BLK, 1), 0)
        r = jnp.where(rows < _B, epad_ref[...], mem_ref[...])
        rn = _normalize_rows(r)
        cols = lax.broadcasted_iota(jnp.int32, (1, _BLK), 1)
        labsrc = jnp.where(cols < _B, labr_ref[0], labm_ref[0])
        dist, m, nn_v, nn = _block_terms(qn_ref[...], rn, labc, labsrc)
        ii = lax.broadcasted_iota(jnp.int32, (_B, 1), 0)
        posm = m & (ii != cols)
        sums[0] += jnp.sum(jnp.where(posm, dist, 0.0))
        sums[1] += jnp.sum(nn_v)
        cnts[0] += jnp.sum(posm.astype(jnp.int32))
        cnts[1] += jnp.sum(nn.astype(jnp.int32))

    @pl.when(pi != 0)
    def _rest():
        rn = _normalize_rows(mem_ref[...])
        dist, m, nn_v, nn = _block_terms(qn_ref[...], rn, labc, labm_ref[0])
        sums[0] += jnp.sum(jnp.where(m, dist, 0.0))
        sums[1] += jnp.sum(nn_v)
        cnts[0] += jnp.sum(m.astype(jnp.int32))
        cnts[1] += jnp.sum(nn.astype(jnp.int32))

    @pl.when(pi == _STEPS - 1)
    def _fin():
        pc = jnp.maximum(cnts[0], 1).astype(jnp.float32)
        nc = jnp.maximum(cnts[1], 1).astype(jnp.float32)
        neg_sum = cnts[1].astype(jnp.float32) - sums[1]
        loss = (sums[0] / pc + neg_sum / nc) * 0.5
        out_ref[...] = jnp.full((1, 1), loss, jnp.float32)


def kernel(embeddings, labels, embedding_memory, label_memory):
    labels = labels.astype(jnp.int32)
    label_memory = label_memory.astype(jnp.int32)

    epad = jnp.zeros((_BLK, _D), jnp.float32).at[:_B].set(embeddings)
    labr = jnp.zeros((_BLK,), jnp.int32).at[:_B].set(labels)
    labr3 = labr.reshape(1, 1, _BLK)
    labc = labels.reshape(_B, 1)
    labm3 = label_memory.reshape(_STEPS, 1, _BLK)

    out = pl.pallas_call(
        _loss_body,
        grid=(_STEPS,),
        in_specs=[
            pl.BlockSpec((_B, _D), lambda i: (0, 0)),
            pl.BlockSpec((_BLK, _D), lambda i: (0, 0)),
            pl.BlockSpec((_B, 1), lambda i: (0, 0)),
            pl.BlockSpec((1, 1, _BLK), lambda i: (0, 0, 0)),
            pl.BlockSpec((_BLK, _D), lambda i: (i, 0)),
            pl.BlockSpec((1, 1, _BLK), lambda i: (i, 0, 0)),
        ],
        out_specs=pl.BlockSpec((1, 1), lambda i: (0, 0)),
        out_shape=jax.ShapeDtypeStruct((1, 1), jnp.float32),
        scratch_shapes=[
            pltpu.VMEM((_B, _D), jnp.float32),
            pltpu.SMEM((2,), jnp.float32),
            pltpu.SMEM((2,), jnp.int32),
        ],
        compiler_params=pltpu.CompilerParams(
            dimension_semantics=("arbitrary",)),
    )(embeddings, epad, labc, labr3, embedding_memory, labm3)
    return out[0, 0]


# speedup vs baseline: 1.7095x; 1.1347x over previous
import jax
import jax.numpy as jnp
from jax import lax
from jax.experimental import pallas as pl
from jax.experimental.pallas import tpu as pltpu

_M = 65536
_B = 1024
_D = 64
_BLK = 2048
_STEPS = _M // _BLK
_HI = lax.Precision.HIGHEST


def _normalize_rows(x):
    s = jnp.sum(x * x, axis=1, keepdims=True)
    return x * lax.rsqrt(jnp.maximum(s, 1e-24))


def _block_terms(qn, rn, labc, labsrc):
    dots = lax.dot_general(qn, rn, (((1,), (1,)), ((), ())),
                           precision=_HI,
                           preferred_element_type=jnp.float32)
    sq = lax.max(2.0 - 2.0 * dots, 1e-12)
    dist = sq * lax.rsqrt(sq)
    mf = jnp.where(labc == labsrc, 1.0, 0.0)
    nnf = jnp.where(dots > 0.5, 1.0, 0.0) * (1.0 - mf)
    return dist, mf, nnf


def _loss_body(emb_ref, epad_ref, labc_ref, labr_ref, mem_ref, labm_ref,
               out_ref, qn_ref, sums):
    pi = pl.program_id(0)
    labc = labc_ref[...]

    @pl.when(pi == 0)
    def _first():
        sums[0] = 0.0
        sums[1] = 0.0
        sums[2] = 0.0
        sums[3] = 0.0
        qn_ref[...] = _normalize_rows(emb_ref[...])
        rows = lax.broadcasted_iota(jnp.int32, (_BLK, 1), 0)
        r = jnp.where(rows < _B, epad_ref[...], mem_ref[...])
        rn = _normalize_rows(r)
        cols = lax.broadcasted_iota(jnp.int32, (1, _BLK), 1)
        labsrc = jnp.where(cols < _B, labr_ref[0], labm_ref[0])
        dist, mf, nnf = _block_terms(qn_ref[...], rn, labc, labsrc)
        ii = lax.broadcasted_iota(jnp.int32, (_B, 1), 0)
        posf = jnp.where(ii != cols, mf, 0.0)
        sums[0] += jnp.sum(posf * dist)
        sums[1] += jnp.sum(nnf * dist)
        sums[2] += jnp.sum(posf)
        sums[3] += jnp.sum(nnf)

    @pl.when(pi != 0)
    def _rest():
        rn = _normalize_rows(mem_ref[...])
        dist, mf, nnf = _block_terms(qn_ref[...], rn, labc, labm_ref[0])
        sums[0] += jnp.sum(mf * dist)
        sums[1] += jnp.sum(nnf * dist)
        sums[2] += jnp.sum(mf)
        sums[3] += jnp.sum(nnf)

    @pl.when(pi == _STEPS - 1)
    def _fin():
        pc = jnp.maximum(sums[2], 1.0)
        nc = jnp.maximum(sums[3], 1.0)
        neg_sum = sums[3] - sums[1]
        loss = (sums[0] / pc + neg_sum / nc) * 0.5
        out_ref[...] = jnp.full((1, 1), loss, jnp.float32)


def kernel(embeddings, labels, embedding_memory, label_memory):
    labels = labels.astype(jnp.int32)
    label_memory = label_memory.astype(jnp.int32)

    epad = jnp.zeros((_BLK, _D), jnp.float32).at[:_B].set(embeddings)
    labr = jnp.zeros((_BLK,), jnp.int32).at[:_B].set(labels)
    labr3 = labr.reshape(1, 1, _BLK)
    labc = labels.reshape(_B, 1)
    labm3 = label_memory.reshape(_STEPS, 1, _BLK)

    out = pl.pallas_call(
        _loss_body,
        grid=(_STEPS,),
        in_specs=[
            pl.BlockSpec((_B, _D), lambda i: (0, 0)),
            pl.BlockSpec((_BLK, _D), lambda i: (0, 0)),
            pl.BlockSpec((_B, 1), lambda i: (0, 0)),
            pl.BlockSpec((1, 1, _BLK), lambda i: (0, 0, 0)),
            pl.BlockSpec((_BLK, _D), lambda i: (i, 0)),
            pl.BlockSpec((1, 1, _BLK), lambda i: (i, 0, 0)),
        ],
        out_specs=pl.BlockSpec((1, 1), lambda i: (0, 0)),
        out_shape=jax.ShapeDtypeStruct((1, 1), jnp.float32),
        scratch_shapes=[
            pltpu.VMEM((_B, _D), jnp.float32),
            pltpu.SMEM((4,), jnp.float32),
        ],
        compiler_params=pltpu.CompilerParams(
            dimension_semantics=("arbitrary",)),
    )(embeddings, epad, labc, labr3, embedding_memory, labm3)
    return out[0, 0]
